# baseline (speedup 1.0000x reference)
import functools

import jax
import jax.numpy as jnp
from jax import lax
from jax.experimental import pallas as pl
from jax.experimental.pallas import tpu as pltpu
from jax.experimental.pallas import tpu_sc as plsc

T = 2048
H = 1024
E = 8
K = 2
F = 512
FS = 512
ROUTE_SCALE = 1.0

BT = 128
G = T * K // BT + E
P = G * BT
TB = 256

NS = 16
NC = 2
TPT = T // NS
A = TPT * K
PW = P // NS
RW = P // (NS * NC)
CW = T // (NS * NC)


def _router_body(x_ref, wg_ref, s_ref):
    logits = jax.lax.dot_general(x_ref[...], wg_ref[...],
                                 (((1,), (1,)), ((), ())),
                                 preferred_element_type=jnp.float32)
    s_ref[...] = jax.nn.sigmoid(logits)


def _shared_body(x_ref, w1s_ref, w2s_ref, out_ref):
    gu = jax.lax.dot_general(x_ref[...], w1s_ref[...], (((1,), (1,)), ((), ())),
                             preferred_element_type=jnp.float32)
    act = jax.nn.silu(gu[:, :FS]) * gu[:, FS:]
    out_ref[...] = jax.lax.dot_general(act, w2s_ref[...],
                                       (((1,), (1,)), ((), ())),
                                       preferred_element_type=jnp.float32)


def _grouped_body(b2e_ref, xs_ref, w1_ref, w2_ref, wsrc_ref, ys_ref):
    gu = jax.lax.dot_general(xs_ref[...].astype(jnp.float32), w1_ref[0],
                             (((1,), (1,)), ((), ())),
                             preferred_element_type=jnp.float32)
    act = jax.nn.silu(gu[:, :F]) * gu[:, F:]
    ye = jax.lax.dot_general(act, w2_ref[0], (((1,), (1,)), ((), ())),
                             preferred_element_type=jnp.float32)
    ys_ref[...] = ye * (wsrc_ref[0, 0][:, None] * ROUTE_SCALE)


def _route_body(s_hbm, bias_hbm, tsrc_hbm, wsrc_hbm, b2e_hbm, pos_hbm,
                sc_l, bias_l, tok_l, ws_l, pos2_l, posP_l, hist_l, hist_all,
                b2e_l, zt_l, zw_l, pk_sh):
    w = lax.axis_index("s")
    active = lax.axis_index("c") == 0

    @pl.when(active)
    def _():
        iota = lax.iota(jnp.int32, 16)
        pltpu.sync_copy(s_hbm.at[pl.ds(w * (TPT * E), TPT * E)], sc_l)
        pltpu.sync_copy(bias_hbm, bias_l)

        for i in range(PW // 16):
            zt_l[pl.ds(i * 16, 16)] = (w * PW + i * 16 + iota) & (T - 1)
            zw_l[pl.ds(i * 16, 16)] = jnp.zeros((16,), jnp.int32)
        pltpu.sync_copy(zt_l, pk_sh.at[pl.ds(w * PW, PW)])
        pltpu.sync_copy(zw_l, pk_sh.at[pl.ds(P + w * PW, PW)])

        id_vecs = [None] * (2 * (TPT // 16))
        for g in range(TPT // 16):
            base = g * (16 * E) + iota * E
            svs = [plsc.load_gather(sc_l, [base + e]) for e in range(E)]
            bvs = [svs[e] + plsc.load_gather(
                bias_l, [jnp.full((16,), e, jnp.int32)]) for e in range(E)]
            m1 = bvs[0]
            for e in range(1, E):
                m1 = jnp.maximum(m1, bvs[e])
            id1 = jnp.full((16,), E, jnp.int32)
            for e in range(E):
                id1 = jnp.minimum(id1, jnp.where(bvs[e] == m1,
                                                 jnp.int32(e), jnp.int32(E)))
            w1v = jnp.zeros((16,), jnp.float32)
            for e in range(E):
                w1v = w1v + jnp.where(id1 == e, svs[e], 0.0)
            bvs2 = [jnp.where(id1 == e, jnp.float32(-3.0e38), bvs[e])
                    for e in range(E)]
            m2 = bvs2[0]
            for e in range(1, E):
                m2 = jnp.maximum(m2, bvs2[e])
            id2 = jnp.full((16,), E, jnp.int32)
            for e in range(E):
                id2 = jnp.minimum(id2, jnp.where(bvs2[e] == m2,
                                                 jnp.int32(e), jnp.int32(E)))
            w2v = jnp.zeros((16,), jnp.float32)
            for e in range(E):
                w2v = w2v + jnp.where(id2 == e, svs[e], 0.0)
            den = jnp.maximum(w1v + w2v, jnp.float32(1e-20))
            id_vecs[g] = id1
            id_vecs[TPT // 16 + g] = id2
            ws_l[pl.ds(g * 16, 16)] = plsc.bitcast(w1v / den, jnp.int32)
            ws_l[pl.ds((TPT + g * 16), 16)] = plsc.bitcast(w2v / den, jnp.int32)
            tok_l[pl.ds(g * 16, 16)] = w * TPT + g * 16 + iota
            tok_l[pl.ds(TPT + g * 16, 16)] = w * TPT + g * 16 + iota

        hvec = jnp.zeros((16,), jnp.int32)
        for e in range(E):
            cnt = jnp.int32(0)
            for j in range(2 * (TPT // 16)):
                cnt = cnt + jnp.sum((id_vecs[j] == e).astype(jnp.int32))
            hvec = hvec + jnp.where(iota == e, cnt, jnp.int32(0))
        hist_l[...] = hvec
        pltpu.sync_copy(hist_l, pk_sh.at[pl.ds(2 * P + w * 16, 16)])
        plsc.subcore_barrier()

        pltpu.sync_copy(pk_sh.at[pl.ds(2 * P, 16 * 16)], hist_all)
        tot = [None] * E
        pref = [None] * E
        for e in range(E):
            col = plsc.load_gather(hist_all, [iota * 16 + e])
            tot[e] = jnp.sum(col)
            pref[e] = jnp.sum(jnp.where(iota < w, col, jnp.int32(0)))
        gs = [None] * E
        acc = jnp.int32(0)
        for e in range(E):
            gs[e] = acc
            aligned = ((tot[e] + (BT - 1)) // BT) * BT
            acc = acc + aligned

        @pl.when(w == 0)
        def _():
            for v in range(3):
                bstart = (v * 16 + iota) * BT
                cntv = jnp.zeros((16,), jnp.int32)
                for e in range(E):
                    cntv = cntv + (gs[e] <= bstart).astype(jnp.int32)
                b2e_l[pl.ds(v * 16, 16)] = cntv - 1
            pltpu.sync_copy(b2e_l, b2e_hbm)

        for e in range(E):
            run = gs[e] + pref[e]
            for j in range(2 * (TPT // 16)):
                m = id_vecs[j] == e
                c = jnp.cumsum(m.astype(jnp.int32))
                posv = run + c - 1
                row = jnp.full((16,), j // (TPT // 16), jnp.int32)
                col = (j % (TPT // 16)) * 16 + iota
                plsc.store_scatter(pos2_l, [row, col], posv, mask=m)
                plsc.store_scatter(posP_l, [row, col], posv + P, mask=m)
                run = run + jnp.sum(m.astype(jnp.int32))

        pltpu.sync_copy(pos2_l.at[0], pos_hbm.at[0, pl.ds(w * TPT, TPT)])
        pltpu.sync_copy(pos2_l.at[1], pos_hbm.at[1, pl.ds(w * TPT, TPT)])
        plsc.subcore_barrier()

        pltpu.sync_copy(tok_l.at[pl.ds(0, TPT)], pk_sh.at[pos2_l.at[0]])
        pltpu.sync_copy(tok_l.at[pl.ds(TPT, TPT)], pk_sh.at[pos2_l.at[1]])
        pltpu.sync_copy(ws_l.at[pl.ds(0, TPT)], pk_sh.at[posP_l.at[0]])
        pltpu.sync_copy(ws_l.at[pl.ds(TPT, TPT)], pk_sh.at[posP_l.at[1]])
        plsc.subcore_barrier()

        pltpu.sync_copy(pk_sh.at[pl.ds(w * PW, PW)], zt_l)
        pltpu.sync_copy(pk_sh.at[pl.ds(P + w * PW, PW)], zw_l)
        pltpu.sync_copy(zt_l, tsrc_hbm.at[pl.ds(w * PW, PW)])
        pltpu.sync_copy(zw_l, wsrc_hbm.at[pl.ds(w * PW, PW)])


def _gather_body(x_hbm, tsrc_hbm, xs_hbm, idx_l, buf0, buf1, sg0, sg1, ss0, ss1):
    wid = lax.axis_index("s") * NC + lax.axis_index("c")
    base = wid * RW
    pltpu.sync_copy(tsrc_hbm.at[pl.ds(base, RW)], idx_l)
    nch = RW // 40
    bufs = [buf0, buf1]
    gsems = [sg0, sg1]
    ssems = [ss0, ss1]
    descs = [None] * nch
    sdescs = [None] * nch
    descs[0] = pltpu.async_copy(x_hbm.at[idx_l.at[pl.ds(0, 40)]], buf0, sg0)
    for c in range(nch):
        b = c % 2
        descs[c].wait()
        if c + 1 < nch:
            if c >= 1:
                sdescs[c - 1].wait()
            descs[c + 1] = pltpu.async_copy(
                x_hbm.at[idx_l.at[pl.ds((c + 1) * 40, 40)]],
                bufs[(c + 1) % 2], gsems[(c + 1) % 2])
        sdescs[c] = pltpu.async_copy(bufs[b], xs_hbm.at[pl.ds(base + c * 40, 40)],
                                     ssems[b])
    sdescs[nch - 2].wait()
    sdescs[nch - 1].wait()


def _combine_body(ys_hbm, pos_hbm, sh_hbm, out_hbm,
                  p0_l, p1_l, b0a, b1a, b2a, b0b, b1b, b2b,
                  s0a, s1a, s2a, s0b, s1b, s2b, ssa, ssb):
    wid = lax.axis_index("s") * NC + lax.axis_index("c")
    base = wid * CW
    pltpu.sync_copy(pos_hbm.at[0, pl.ds(base, CW)], p0_l)
    pltpu.sync_copy(pos_hbm.at[1, pl.ds(base, CW)], p1_l)
    nch = CW // 16
    B0 = [b0a, b0b]
    B1 = [b1a, b1b]
    B2 = [b2a, b2b]
    S0 = [s0a, s0b]
    S1 = [s1a, s1b]
    S2 = [s2a, s2b]
    SS = [ssa, ssb]

    def start(c):
        s = c % 2
        return (pltpu.async_copy(ys_hbm.at[p0_l.at[pl.ds(c * 16, 16)]], B0[s], S0[s]),
                pltpu.async_copy(ys_hbm.at[p1_l.at[pl.ds(c * 16, 16)]], B1[s], S1[s]),
                pltpu.async_copy(sh_hbm.at[pl.ds(base + c * 16, 16)], B2[s], S2[s]))

    descs = [None] * nch
    sd = [None] * nch
    descs[0] = start(0)
    for c in range(nch):
        s = c % 2
        if c + 1 < nch:
            if c >= 1:
                sd[c - 1].wait()
            descs[c + 1] = start(c + 1)
        for dsc in descs[c]:
            dsc.wait()
        b0, b1, b2 = B0[s], B1[s], B2[s]

        def add_body(j, _):
            sl = pl.ds(j * 16, 16)
            for r in range(16):
                b0[r, sl] = b0[r, sl] + b1[r, sl] + b2[r, sl]
            return 0

        lax.fori_loop(0, H // 16, add_body, 0)
        sd[c] = pltpu.async_copy(b0, out_hbm.at[pl.ds(base + c * 16, 16)], SS[s])
    sd[nch - 2].wait()
    sd[nch - 1].wait()


@jax.jit
def kernel(hidden_states, W_gate, expert_bias, w1, w2, w1_shared, w2_shared):
    x = hidden_states
    f32 = jnp.float32
    i32 = jnp.int32

    scores = pl.pallas_call(
        _router_body,
        grid=(1,),
        in_specs=[pl.BlockSpec((T, H), lambda i: (0, 0)),
                  pl.BlockSpec((E, H), lambda i: (0, 0))],
        out_specs=pl.BlockSpec((T, E), lambda i: (0, 0)),
        out_shape=jax.ShapeDtypeStruct((T, E), f32),
    )(x, W_gate)

    bias16 = jnp.zeros((16,), f32).at[:E].set(expert_bias)

    mesh = plsc.VectorSubcoreMesh(core_axis_name="c", subcore_axis_name="s",
                                  num_cores=NC, num_subcores=NS)
    route = pl.kernel(
        _route_body,
        out_type=[jax.ShapeDtypeStruct((P,), i32),
                  jax.ShapeDtypeStruct((P,), i32),
                  jax.ShapeDtypeStruct((48,), i32),
                  jax.ShapeDtypeStruct((K, T), i32)],
        mesh=mesh,
        compiler_params=pltpu.CompilerParams(needs_layout_passes=False),
        scratch_types=[pltpu.VMEM((TPT * E,), f32),
                       pltpu.VMEM((16,), f32),
                       pltpu.VMEM((A,), i32),
                       pltpu.VMEM((A,), i32),
                       pltpu.VMEM((K, TPT), i32),
                       pltpu.VMEM((K, TPT), i32),
                       pltpu.VMEM((16,), i32),
                       pltpu.VMEM((256,), i32),
                       pltpu.VMEM((48,), i32),
                       pltpu.VMEM((PW,), i32),
                       pltpu.VMEM((PW,), i32),
                       pltpu.VMEM_SHARED((2 * P + 256,), i32)],
    )
    tsrc, wsrc_i, b2e, pos = route(scores.reshape(T * E), bias16)
    wsrc = jax.lax.bitcast_convert_type(wsrc_i, f32)

    xv = jax.lax.bitcast_convert_type(
        x.astype(jnp.bfloat16).reshape(T, H // 2, 2), jnp.uint32)
    xs_u = pl.kernel(
        _gather_body,
        out_type=jax.ShapeDtypeStruct((P, H // 2), jnp.uint32),
        mesh=mesh,
        compiler_params=pltpu.CompilerParams(needs_layout_passes=False),
        scratch_types=[pltpu.VMEM((RW,), i32),
                       pltpu.VMEM((40, H // 2), jnp.uint32),
                       pltpu.VMEM((40, H // 2), jnp.uint32),
                       pltpu.SemaphoreType.DMA,
                       pltpu.SemaphoreType.DMA,
                       pltpu.SemaphoreType.DMA,
                       pltpu.SemaphoreType.DMA],
    )(xv, tsrc)
    xs = jax.lax.bitcast_convert_type(xs_u, jnp.bfloat16).reshape(P, H)

    shared_out = pl.pallas_call(
        _shared_body,
        grid=(T // TB,),
        in_specs=[pl.BlockSpec((TB, H), lambda i: (i, 0)),
                  pl.BlockSpec((2 * FS, H), lambda i: (0, 0)),
                  pl.BlockSpec((H, FS), lambda i: (0, 0))],
        out_specs=pl.BlockSpec((TB, H), lambda i: (i, 0)),
        out_shape=jax.ShapeDtypeStruct((T, H), f32),
    )(x, w1_shared, w2_shared)

    wsrc3 = wsrc.reshape(G, 1, BT)
    ys = pl.pallas_call(
        _grouped_body,
        grid_spec=pltpu.PrefetchScalarGridSpec(
            num_scalar_prefetch=1,
            grid=(G,),
            in_specs=[
                pl.BlockSpec((BT, H), lambda i, b2e_ref: (i, 0)),
                pl.BlockSpec((1, 2 * F, H), lambda i, b2e_ref: (b2e_ref[i], 0, 0)),
                pl.BlockSpec((1, H, F), lambda i, b2e_ref: (b2e_ref[i], 0, 0)),
                pl.BlockSpec((1, 1, BT), lambda i, b2e_ref: (i, 0, 0)),
            ],
            out_specs=pl.BlockSpec((BT, H), lambda i, b2e_ref: (i, 0)),
        ),
        out_shape=jax.ShapeDtypeStruct((P, H), f32),
    )(b2e, xs, w1, w2, wsrc3)

    final = pl.kernel(
        _combine_body,
        out_type=jax.ShapeDtypeStruct((T, H), f32),
        mesh=mesh,
        compiler_params=pltpu.CompilerParams(needs_layout_passes=False),
        scratch_types=([pltpu.VMEM((CW,), i32)] * 2
                       + [pltpu.VMEM((16, H), f32)] * 6
                       + [pltpu.SemaphoreType.DMA] * 8),
    )(ys, pos, shared_out)

    return final


# speedup vs baseline: 1.8949x; 1.8949x over previous
import functools

import jax
import jax.numpy as jnp
from jax import lax
from jax.experimental import pallas as pl
from jax.experimental.pallas import tpu as pltpu
from jax.experimental.pallas import tpu_sc as plsc

T = 2048
H = 1024
E = 8
K = 2
F = 512
FS = 512
ROUTE_SCALE = 1.0

BT = 128
G = T * K // BT + E
P = G * BT
TB = 256

NS = 16
NC = 2
TPT = T // NS
A = TPT * K
PW = P // NS
RW = P // (NS * NC)
CW = T // (NS * NC)


def _router_body(x_ref, wg_ref, s_ref):
    logits = jax.lax.dot_general(x_ref[...], wg_ref[...],
                                 (((1,), (1,)), ((), ())),
                                 preferred_element_type=jnp.float32)
    s_ref[...] = jax.nn.sigmoid(logits)


def _shared_body(x_ref, w1s_ref, w2s_ref, out_ref):
    gu = jax.lax.dot_general(x_ref[...], w1s_ref[...], (((1,), (1,)), ((), ())),
                             preferred_element_type=jnp.float32)
    act = jax.nn.silu(gu[:, :FS]) * gu[:, FS:]
    out_ref[...] = jax.lax.dot_general(act, w2s_ref[...],
                                       (((1,), (1,)), ((), ())),
                                       preferred_element_type=jnp.float32)


def _grouped_body(b2e_ref, xs_ref, w1_ref, w2_ref, wsrc_ref, ys_ref):
    gu = jax.lax.dot_general(xs_ref[...], w1_ref[0], (((1,), (1,)), ((), ())),
                             preferred_element_type=jnp.float32)
    act = jax.nn.silu(gu[:, :F]) * gu[:, F:]
    ye = jax.lax.dot_general(act, w2_ref[0], (((1,), (1,)), ((), ())),
                             preferred_element_type=jnp.float32)
    ys_ref[...] = ye * (wsrc_ref[0, 0][:, None] * ROUTE_SCALE)


def _route_body(s_hbm, bias_hbm, tsrc_hbm, wsrc_hbm, b2e_hbm, pos_hbm,
                sc_l, bias_l, tok_l, ws_l, pos2_l, posP_l, hist_l, hist_all,
                b2e_l, zt_l, zw_l, pk_sh):
    w = lax.axis_index("s")
    active = lax.axis_index("c") == 0

    @pl.when(active)
    def _():
        iota = lax.iota(jnp.int32, 16)
        pltpu.sync_copy(s_hbm.at[pl.ds(w * (TPT * E), TPT * E)], sc_l)
        pltpu.sync_copy(bias_hbm, bias_l)

        for i in range(PW // 16):
            zt_l[pl.ds(i * 16, 16)] = (w * PW + i * 16 + iota) & (T - 1)
            zw_l[pl.ds(i * 16, 16)] = jnp.zeros((16,), jnp.int32)
        pltpu.sync_copy(zt_l, pk_sh.at[pl.ds(w * PW, PW)])
        pltpu.sync_copy(zw_l, pk_sh.at[pl.ds(P + w * PW, PW)])

        id_vecs = [None] * (2 * (TPT // 16))
        for g in range(TPT // 16):
            base = g * (16 * E) + iota * E
            svs = [plsc.load_gather(sc_l, [base + e]) for e in range(E)]
            bvs = [svs[e] + plsc.load_gather(
                bias_l, [jnp.full((16,), e, jnp.int32)]) for e in range(E)]
            m1 = bvs[0]
            for e in range(1, E):
                m1 = jnp.maximum(m1, bvs[e])
            id1 = jnp.full((16,), E, jnp.int32)
            for e in range(E):
                id1 = jnp.minimum(id1, jnp.where(bvs[e] == m1,
                                                 jnp.int32(e), jnp.int32(E)))
            w1v = jnp.zeros((16,), jnp.float32)
            for e in range(E):
                w1v = w1v + jnp.where(id1 == e, svs[e], 0.0)
            bvs2 = [jnp.where(id1 == e, jnp.float32(-3.0e38), bvs[e])
                    for e in range(E)]
            m2 = bvs2[0]
            for e in range(1, E):
                m2 = jnp.maximum(m2, bvs2[e])
            id2 = jnp.full((16,), E, jnp.int32)
            for e in range(E):
                id2 = jnp.minimum(id2, jnp.where(bvs2[e] == m2,
                                                 jnp.int32(e), jnp.int32(E)))
            w2v = jnp.zeros((16,), jnp.float32)
            for e in range(E):
                w2v = w2v + jnp.where(id2 == e, svs[e], 0.0)
            den = jnp.maximum(w1v + w2v, jnp.float32(1e-20))
            id_vecs[g] = id1
            id_vecs[TPT // 16 + g] = id2
            ws_l[pl.ds(g * 16, 16)] = plsc.bitcast(w1v / den, jnp.int32)
            ws_l[pl.ds((TPT + g * 16), 16)] = plsc.bitcast(w2v / den, jnp.int32)
            tok_l[pl.ds(g * 16, 16)] = w * TPT + g * 16 + iota
            tok_l[pl.ds(TPT + g * 16, 16)] = w * TPT + g * 16 + iota

        hvec = jnp.zeros((16,), jnp.int32)
        for e in range(E):
            cnt = jnp.int32(0)
            for j in range(2 * (TPT // 16)):
                cnt = cnt + jnp.sum((id_vecs[j] == e).astype(jnp.int32))
            hvec = hvec + jnp.where(iota == e, cnt, jnp.int32(0))
        hist_l[...] = hvec
        pltpu.sync_copy(hist_l, pk_sh.at[pl.ds(2 * P + w * 16, 16)])
        plsc.subcore_barrier()

        pltpu.sync_copy(pk_sh.at[pl.ds(2 * P, 16 * 16)], hist_all)
        tot = [None] * E
        pref = [None] * E
        for e in range(E):
            col = plsc.load_gather(hist_all, [iota * 16 + e])
            tot[e] = jnp.sum(col)
            pref[e] = jnp.sum(jnp.where(iota < w, col, jnp.int32(0)))
        gs = [None] * E
        acc = jnp.int32(0)
        for e in range(E):
            gs[e] = acc
            aligned = ((tot[e] + (BT - 1)) // BT) * BT
            acc = acc + aligned

        @pl.when(w == 0)
        def _():
            for v in range(3):
                bstart = (v * 16 + iota) * BT
                cntv = jnp.zeros((16,), jnp.int32)
                for e in range(E):
                    cntv = cntv + (gs[e] <= bstart).astype(jnp.int32)
                b2e_l[pl.ds(v * 16, 16)] = cntv - 1
            pltpu.sync_copy(b2e_l, b2e_hbm)

        for e in range(E):
            run = gs[e] + pref[e]
            for j in range(2 * (TPT // 16)):
                m = id_vecs[j] == e
                c = jnp.cumsum(m.astype(jnp.int32))
                posv = run + c - 1
                row = jnp.full((16,), j // (TPT // 16), jnp.int32)
                col = (j % (TPT // 16)) * 16 + iota
                plsc.store_scatter(pos2_l, [row, col], posv, mask=m)
                plsc.store_scatter(posP_l, [row, col], posv + P, mask=m)
                run = run + jnp.sum(m.astype(jnp.int32))

        pltpu.sync_copy(pos2_l.at[0], pos_hbm.at[0, pl.ds(w * TPT, TPT)])
        pltpu.sync_copy(pos2_l.at[1], pos_hbm.at[1, pl.ds(w * TPT, TPT)])
        plsc.subcore_barrier()

        pltpu.sync_copy(tok_l.at[pl.ds(0, TPT)], pk_sh.at[pos2_l.at[0]])
        pltpu.sync_copy(tok_l.at[pl.ds(TPT, TPT)], pk_sh.at[pos2_l.at[1]])
        pltpu.sync_copy(ws_l.at[pl.ds(0, TPT)], pk_sh.at[posP_l.at[0]])
        pltpu.sync_copy(ws_l.at[pl.ds(TPT, TPT)], pk_sh.at[posP_l.at[1]])
        plsc.subcore_barrier()

        pltpu.sync_copy(pk_sh.at[pl.ds(w * PW, PW)], zt_l)
        pltpu.sync_copy(pk_sh.at[pl.ds(P + w * PW, PW)], zw_l)
        pltpu.sync_copy(zt_l, tsrc_hbm.at[pl.ds(w * PW, PW)])
        pltpu.sync_copy(zw_l, wsrc_hbm.at[pl.ds(w * PW, PW)])


def _gather_body(x_hbm, tsrc_hbm, xs_hbm, idx_l, buf0, buf1, sg0, sg1, ss0, ss1):
    wid = lax.axis_index("s") * NC + lax.axis_index("c")
    base = wid * RW
    pltpu.sync_copy(tsrc_hbm.at[pl.ds(base, RW)], idx_l)
    nch = RW // 40
    bufs = [buf0, buf1]
    gsems = [sg0, sg1]
    ssems = [ss0, ss1]
    descs = [None] * nch
    sdescs = [None] * nch
    descs[0] = pltpu.async_copy(x_hbm.at[idx_l.at[pl.ds(0, 40)]], buf0, sg0)
    for c in range(nch):
        b = c % 2
        descs[c].wait()
        if c + 1 < nch:
            if c >= 1:
                sdescs[c - 1].wait()
            descs[c + 1] = pltpu.async_copy(
                x_hbm.at[idx_l.at[pl.ds((c + 1) * 40, 40)]],
                bufs[(c + 1) % 2], gsems[(c + 1) % 2])
        sdescs[c] = pltpu.async_copy(bufs[b], xs_hbm.at[pl.ds(base + c * 40, 40)],
                                     ssems[b])
    sdescs[nch - 2].wait()
    sdescs[nch - 1].wait()


def _combine_body(ys_hbm, pos_hbm, sh_hbm, out_hbm,
                  p0_l, p1_l, b0a, b1a, b2a, b0b, b1b, b2b,
                  s0a, s1a, s2a, s0b, s1b, s2b, ssa, ssb):
    wid = lax.axis_index("s") * NC + lax.axis_index("c")
    base = wid * CW
    pltpu.sync_copy(pos_hbm.at[0, pl.ds(base, CW)], p0_l)
    pltpu.sync_copy(pos_hbm.at[1, pl.ds(base, CW)], p1_l)
    nch = CW // 16
    B0 = [b0a, b0b]
    B1 = [b1a, b1b]
    B2 = [b2a, b2b]
    S0 = [s0a, s0b]
    S1 = [s1a, s1b]
    S2 = [s2a, s2b]
    SS = [ssa, ssb]

    def start(c):
        s = c % 2
        return (pltpu.async_copy(ys_hbm.at[p0_l.at[pl.ds(c * 16, 16)]], B0[s], S0[s]),
                pltpu.async_copy(ys_hbm.at[p1_l.at[pl.ds(c * 16, 16)]], B1[s], S1[s]),
                pltpu.async_copy(sh_hbm.at[pl.ds(base + c * 16, 16)], B2[s], S2[s]))

    descs = [None] * nch
    sd = [None] * nch
    descs[0] = start(0)
    for c in range(nch):
        s = c % 2
        if c + 1 < nch:
            if c >= 1:
                sd[c - 1].wait()
            descs[c + 1] = start(c + 1)
        for dsc in descs[c]:
            dsc.wait()
        b0, b1, b2 = B0[s], B1[s], B2[s]

        def add_body(j, _):
            sl = pl.ds(j * 16, 16)
            for r in range(16):
                b0[r, sl] = b0[r, sl] + b1[r, sl] + b2[r, sl]
            return 0

        lax.fori_loop(0, H // 16, add_body, 0)
        sd[c] = pltpu.async_copy(b0, out_hbm.at[pl.ds(base + c * 16, 16)], SS[s])
    sd[nch - 2].wait()
    sd[nch - 1].wait()


@jax.jit
def kernel(hidden_states, W_gate, expert_bias, w1, w2, w1_shared, w2_shared):
    x = hidden_states
    f32 = jnp.float32
    i32 = jnp.int32

    scores = pl.pallas_call(
        _router_body,
        grid=(1,),
        in_specs=[pl.BlockSpec((T, H), lambda i: (0, 0)),
                  pl.BlockSpec((E, H), lambda i: (0, 0))],
        out_specs=pl.BlockSpec((T, E), lambda i: (0, 0)),
        out_shape=jax.ShapeDtypeStruct((T, E), f32),
    )(x, W_gate)

    bias16 = jnp.zeros((16,), f32).at[:E].set(expert_bias)

    mesh = plsc.VectorSubcoreMesh(core_axis_name="c", subcore_axis_name="s",
                                  num_cores=NC, num_subcores=NS)
    route = pl.kernel(
        _route_body,
        out_type=[jax.ShapeDtypeStruct((P,), i32),
                  jax.ShapeDtypeStruct((P,), i32),
                  jax.ShapeDtypeStruct((48,), i32),
                  jax.ShapeDtypeStruct((K, T), i32)],
        mesh=mesh,
        compiler_params=pltpu.CompilerParams(needs_layout_passes=False),
        scratch_types=[pltpu.VMEM((TPT * E,), f32),
                       pltpu.VMEM((16,), f32),
                       pltpu.VMEM((A,), i32),
                       pltpu.VMEM((A,), i32),
                       pltpu.VMEM((K, TPT), i32),
                       pltpu.VMEM((K, TPT), i32),
                       pltpu.VMEM((16,), i32),
                       pltpu.VMEM((256,), i32),
                       pltpu.VMEM((48,), i32),
                       pltpu.VMEM((PW,), i32),
                       pltpu.VMEM((PW,), i32),
                       pltpu.VMEM_SHARED((2 * P + 256,), i32)],
    )
    tsrc, wsrc_i, b2e, pos = route(scores.reshape(T * E), bias16)
    wsrc = jax.lax.bitcast_convert_type(wsrc_i, f32)

    xs = pl.kernel(
        _gather_body,
        out_type=jax.ShapeDtypeStruct((P, H), f32),
        mesh=mesh,
        compiler_params=pltpu.CompilerParams(needs_layout_passes=False),
        scratch_types=[pltpu.VMEM((RW,), i32),
                       pltpu.VMEM((40, H), f32),
                       pltpu.VMEM((40, H), f32),
                       pltpu.SemaphoreType.DMA,
                       pltpu.SemaphoreType.DMA,
                       pltpu.SemaphoreType.DMA,
                       pltpu.SemaphoreType.DMA],
    )(x, tsrc)

    shared_out = pl.pallas_call(
        _shared_body,
        grid=(T // TB,),
        in_specs=[pl.BlockSpec((TB, H), lambda i: (i, 0)),
                  pl.BlockSpec((2 * FS, H), lambda i: (0, 0)),
                  pl.BlockSpec((H, FS), lambda i: (0, 0))],
        out_specs=pl.BlockSpec((TB, H), lambda i: (i, 0)),
        out_shape=jax.ShapeDtypeStruct((T, H), f32),
    )(x, w1_shared, w2_shared)

    wsrc3 = wsrc.reshape(G, 1, BT)
    ys = pl.pallas_call(
        _grouped_body,
        grid_spec=pltpu.PrefetchScalarGridSpec(
            num_scalar_prefetch=1,
            grid=(G,),
            in_specs=[
                pl.BlockSpec((BT, H), lambda i, b2e_ref: (i, 0)),
                pl.BlockSpec((1, 2 * F, H), lambda i, b2e_ref: (b2e_ref[i], 0, 0)),
                pl.BlockSpec((1, H, F), lambda i, b2e_ref: (b2e_ref[i], 0, 0)),
                pl.BlockSpec((1, 1, BT), lambda i, b2e_ref: (i, 0, 0)),
            ],
            out_specs=pl.BlockSpec((BT, H), lambda i, b2e_ref: (i, 0)),
        ),
        out_shape=jax.ShapeDtypeStruct((P, H), f32),
    )(b2e, xs, w1, w2, wsrc3)

    final = pl.kernel(
        _combine_body,
        out_type=jax.ShapeDtypeStruct((T, H), f32),
        mesh=mesh,
        compiler_params=pltpu.CompilerParams(needs_layout_passes=False),
        scratch_types=([pltpu.VMEM((CW,), i32)] * 2
                       + [pltpu.VMEM((16, H), f32)] * 6
                       + [pltpu.SemaphoreType.DMA] * 8),
    )(ys, pos, shared_out)

    return final
